# baseline (speedup 1.0000x reference)
import functools
import math

import jax
import jax.numpy as jnp
from jax import lax
from jax.experimental import pallas as pl
from jax.experimental.pallas import tpu as pltpu, tpu_sc as plsc

L = 2048
D_MODEL = 768
N_HEADS = 12
HEAD_DIM = D_MODEL // N_HEADS
D_C = 128
D_FF = 1024
N_EXP = 8
MAX_OUT = 5
DEC_PAD = 8
THETA = 10000.0
EPS = 1e-6
F32 = jnp.float32


def _rms(x, w):
    return x * jax.lax.rsqrt(jnp.mean(x * x, axis=-1, keepdims=True) + EPS) * w


def _linear(x, w, b=None, res=None, norm=None, bm=None):
    M, K = x.shape
    N = w.shape[0]
    if bm is None:
        bm = 256 if M % 256 == 0 else M
    has_b, has_res, has_norm = b is not None, res is not None, norm is not None

    def kfn(*refs):
        idx = 2
        xv = refs[0][...]
        wv = refs[1][...]
        if has_norm:
            xv = _rms(xv, refs[idx][...])
            idx += 1
        y = jnp.dot(xv, wv.T, preferred_element_type=F32)
        if has_b:
            y = y + refs[idx][...]
            idx += 1
        if has_res:
            y = y + refs[idx][...]
            idx += 1
        refs[idx][...] = y

    inputs = [x, w]
    specs = [pl.BlockSpec((bm, K), lambda i: (i, 0)),
             pl.BlockSpec((N, K), lambda i: (0, 0))]
    if has_norm:
        inputs.append(norm.reshape(1, K))
        specs.append(pl.BlockSpec((1, K), lambda i: (0, 0)))
    if has_b:
        inputs.append(b.reshape(1, N))
        specs.append(pl.BlockSpec((1, N), lambda i: (0, 0)))
    if has_res:
        inputs.append(res)
        specs.append(pl.BlockSpec((bm, N), lambda i: (i, 0)))
    return pl.pallas_call(
        kfn,
        grid=(M // bm,),
        in_specs=specs,
        out_specs=pl.BlockSpec((bm, N), lambda i: (i, 0)),
        out_shape=jax.ShapeDtypeStruct((M, N), F32),
    )(*inputs)


def _attn_rope(q_r, k_r, k_c, v_c, ts):
    H, Lq, Dh = q_r.shape
    scale = Dh ** -0.5

    def kfn(ts_ref, q_ref, kr_ref, kc_ref, vc_ref, o_ref):
        half = Dh // 2
        idx = jax.lax.broadcasted_iota(jnp.int32, (1, half), 1).astype(F32)
        freqs = jnp.exp(-(math.log(THETA) * 2.0 / Dh) * idx)
        t = ts_ref[...] * freqs
        cs = jnp.cos(jnp.concatenate([t, t], axis=-1))

        def rope(v):
            rot = jnp.concatenate([-v[:, half:], v[:, :half]], axis=-1)
            return (v + rot) * cs

        q = rope(q_ref[0])
        ks = kc_ref[0] + rope(kr_ref[0])
        s = jnp.dot(q * scale, ks.T, preferred_element_type=F32)
        m = jnp.max(s, axis=-1, keepdims=True)
        e = jnp.exp(s - m)
        p = e / jnp.sum(e, axis=-1, keepdims=True)
        o_ref[0] = jnp.dot(p, vc_ref[0], preferred_element_type=F32)

    blk = pl.BlockSpec((1, Lq, Dh), lambda h: (h, 0, 0))
    return pl.pallas_call(
        kfn,
        grid=(H,),
        in_specs=[pl.BlockSpec((Lq, 1), lambda h: (0, 0)), blk, blk, blk, blk],
        out_specs=blk,
        out_shape=jax.ShapeDtypeStruct((H, Lq, Dh), F32),
    )(ts, q_r, k_r, k_c, v_c)


def _attn_plain(q, k, v, n_valid):
    H, Lq, Dh = q.shape
    Lk = k.shape[1]
    scale = Dh ** -0.5

    def kfn(q_ref, k_ref, v_ref, o_ref):
        s = jnp.dot(q_ref[0] * scale, k_ref[0].T, preferred_element_type=F32)
        if n_valid < Lk:
            kid = jax.lax.broadcasted_iota(jnp.int32, (Lq, Lk), 1)
            s = jnp.where(kid < n_valid, s, -1e30)
        m = jnp.max(s, axis=-1, keepdims=True)
        e = jnp.exp(s - m)
        p = e / jnp.sum(e, axis=-1, keepdims=True)
        o_ref[0] = jnp.dot(p, v_ref[0], preferred_element_type=F32)

    qblk = pl.BlockSpec((1, Lq, Dh), lambda h: (h, 0, 0))
    kblk = pl.BlockSpec((1, Lk, Dh), lambda h: (h, 0, 0))
    return pl.pallas_call(
        kfn,
        grid=(H,),
        in_specs=[qblk, kblk, kblk],
        out_specs=qblk,
        out_shape=jax.ShapeDtypeStruct((H, Lq, Dh), F32),
    )(q, k, v)


def _moe(x, norm, gate, bias, w1, w2, bm=None):
    M, D = x.shape
    E, F2, _ = w1.shape
    F = F2 // 2
    if bm is None:
        bm = 256 if M % 256 == 0 else M

    def kfn(x_ref, n_ref, g_ref, b_ref, w1_ref, w2_ref, o_ref):
        e = pl.program_id(1)
        xv = x_ref[...]
        xn = _rms(xv, n_ref[...])
        logits = jnp.dot(xn, g_ref[...].T, preferred_element_type=F32) + b_ref[...]
        lm = jnp.max(logits, axis=-1, keepdims=True)
        ex = jnp.exp(logits - lm)
        probs = ex / jnp.sum(ex, axis=-1, keepdims=True)
        i1 = jnp.argmax(probs, axis=-1)
        m1 = jnp.max(probs, axis=-1)
        eid = jax.lax.broadcasted_iota(jnp.int32, probs.shape, 1)
        masked = jnp.where(eid == i1[:, None], -1.0, probs)
        i2 = jnp.argmax(masked, axis=-1)
        m2 = jnp.max(masked, axis=-1)
        wsum = m1 + m2
        we = (jnp.where(i1 == e, m1, 0.0) + jnp.where(i2 == e, m2, 0.0)) / wsum

        h = jnp.dot(xn, w1_ref[0].T, preferred_element_type=F32)
        a = h[:, :F]
        b2 = h[:, F:]
        g = a * jax.nn.sigmoid(a) * b2
        o = jnp.dot(g, w2_ref[0].T, preferred_element_type=F32)

        @pl.when(e == 0)
        def _():
            o_ref[...] = xv

        o_ref[...] += we[:, None] * o

    return pl.pallas_call(
        kfn,
        grid=(M // bm, E),
        in_specs=[
            pl.BlockSpec((bm, D), lambda i, e: (i, 0)),
            pl.BlockSpec((1, D), lambda i, e: (0, 0)),
            pl.BlockSpec((E, D), lambda i, e: (0, 0)),
            pl.BlockSpec((1, E), lambda i, e: (0, 0)),
            pl.BlockSpec((1, F2, D), lambda i, e: (e, 0, 0)),
            pl.BlockSpec((1, D, F), lambda i, e: (e, 0, 0)),
        ],
        out_specs=pl.BlockSpec((bm, D), lambda i, e: (i, 0)),
        out_shape=jax.ShapeDtypeStruct((M, D), F32),
    )(x, norm.reshape(1, D), gate, bias.reshape(1, E), w1, w2)


BM_G = 128
NROWS = 2 * L + N_EXP * BM_G
NB = NROWS // BM_G
SC_NW = 32
SC_CH = 64


def _route(h, norm, gate, bias):
    M, D = h.shape
    E = N_EXP
    BLK = 512
    NBLK = 2 * M // BLK

    def kfn(h_ref, n_ref, g_ref, b_ref, xn_ref, dst_ref, ws_ref, be_ref):
        xv = h_ref[...]
        xn = _rms(xv, n_ref[...])
        xn_ref[...] = xn
        logits = jnp.dot(xn, g_ref[...].T, preferred_element_type=F32, precision=lax.Precision.HIGHEST) + b_ref[...]
        lm = jnp.max(logits, axis=-1, keepdims=True)
        ex = jnp.exp(logits - lm)
        probs = ex / jnp.sum(ex, axis=-1, keepdims=True)
        i1 = jnp.argmax(probs, axis=-1)
        m1 = jnp.max(probs, axis=-1, keepdims=True)
        eid = jax.lax.broadcasted_iota(jnp.int32, probs.shape, 1)
        masked = jnp.where(eid == i1[:, None], -1.0, probs)
        i2 = jnp.argmax(masked, axis=-1)
        m2 = jnp.max(masked, axis=-1, keepdims=True)
        wsum = m1 + m2
        ws_ref[...] = jnp.concatenate([m1 / wsum, m2 / wsum], axis=1)

        one1 = (eid == i1[:, None]).astype(F32)
        one2 = (eid == i2[:, None]).astype(F32)
        ri = jax.lax.broadcasted_iota(jnp.int32, (BLK, BLK), 0)
        ci = jax.lax.broadcasted_iota(jnp.int32, (BLK, BLK), 1)
        tri = (ci < ri).astype(F32)
        ones_e = jnp.ones((E, 1), F32)

        carry = jnp.zeros((1, E), F32)
        ranks, onehots = [], []
        for one in (one1, one2):
            for b in range(M // BLK):
                ob = one[b * BLK:(b + 1) * BLK]
                r = jnp.dot(tri, ob, preferred_element_type=F32, precision=lax.Precision.HIGHEST) + carry
                ranks.append(jnp.dot(r * ob, ones_e,
                                     preferred_element_type=F32, precision=lax.Precision.HIGHEST))
                onehots.append(ob)
                carry = carry + jnp.sum(ob, axis=0, keepdims=True)

        counts = carry.astype(jnp.int32)
        padded = ((counts + BM_G - 1) // BM_G) * BM_G
        padded_f = padded.astype(F32)
        ji = jax.lax.broadcasted_iota(jnp.int32, (E, E), 0)
        ei = jax.lax.broadcasted_iota(jnp.int32, (E, E), 1)
        u_strict = (ji < ei).astype(F32)
        u_incl = (ji <= ei).astype(F32)
        rstart = jnp.dot(padded_f, u_strict, preferred_element_type=F32, precision=lax.Precision.HIGHEST)
        cumb = jnp.dot(padded_f / BM_G, u_incl, preferred_element_type=F32, precision=lax.Precision.HIGHEST)

        cols = []
        for ob, rk in zip(onehots, ranks):
            rsel = jnp.dot(ob * rstart, ones_e, preferred_element_type=F32, precision=lax.Precision.HIGHEST)
            cols.append((rsel + rk).astype(jnp.int32))
        dst_ref[...] = jnp.concatenate(cols, axis=1)

        bix = jax.lax.broadcasted_iota(jnp.int32, (NB, 1), 0).astype(F32)
        ge = (bix >= cumb).astype(F32)
        nbefore = jnp.dot(ge, ones_e, preferred_element_type=F32, precision=lax.Precision.HIGHEST)
        be_ref[...] = jnp.minimum(nbefore, E - 1).astype(jnp.int32)

    full = lambda shape: pl.BlockSpec(shape, lambda: tuple(0 for _ in shape))
    return pl.pallas_call(
        kfn,
        in_specs=[full((M, D)), full((1, D)), full((E, D)), full((1, E))],
        out_specs=[full((M, D)), full((BLK, NBLK)), full((M, 2)), full((NB, 1))],
        out_shape=[
            jax.ShapeDtypeStruct((M, D), F32),
            jax.ShapeDtypeStruct((BLK, NBLK), jnp.int32),
            jax.ShapeDtypeStruct((M, 2), F32),
            jax.ShapeDtypeStruct((NB, 1), jnp.int32),
        ],
    )(h, norm.reshape(1, D), gate, bias.reshape(1, E))


def _sc_scatter_rows(xn, dst):
    M, D = xn.shape
    mesh = plsc.VectorSubcoreMesh(core_axis_name="c", subcore_axis_name="s")
    n_ch = 2 * M // (SC_NW * SC_CH)

    @functools.partial(
        pl.kernel, mesh=mesh,
        out_type=jax.ShapeDtypeStruct((NROWS, D), F32),
        scratch_types=[
            pltpu.VMEM((SC_CH,), jnp.int32),
            pltpu.VMEM((SC_CH, D), F32),
            pltpu.SemaphoreType.DMA,
        ],
    )
    def k(xn_hbm, dst_hbm, rows_hbm, idx_v, buf_v, sem):
        wid = lax.axis_index("s") * 2 + lax.axis_index("c")
        for c in range(n_ch):
            i0 = (wid * n_ch + c) * SC_CH
            tok0 = lax.rem(i0, M)
            pltpu.sync_copy(dst_hbm.at[pl.ds(i0, SC_CH)], idx_v)
            pltpu.sync_copy(xn_hbm.at[pl.ds(tok0, SC_CH)], buf_v)
            pltpu.async_copy(buf_v, rows_hbm.at[idx_v], sem).wait()

    return k(xn, dst)


def _sc_gather_rows(rows, dst):
    NR, D = rows.shape
    M2 = dst.shape[0]
    mesh = plsc.VectorSubcoreMesh(core_axis_name="c", subcore_axis_name="s")
    n_ch = M2 // (SC_NW * SC_CH)

    @functools.partial(
        pl.kernel, mesh=mesh,
        out_type=jax.ShapeDtypeStruct((M2, D), F32),
        scratch_types=[
            pltpu.VMEM((SC_CH,), jnp.int32),
            pltpu.VMEM((SC_CH, D), F32),
            pltpu.SemaphoreType.DMA,
        ],
    )
    def k(rows_hbm, dst_hbm, oo_hbm, idx_v, buf_v, sem):
        wid = lax.axis_index("s") * 2 + lax.axis_index("c")
        for c in range(n_ch):
            i0 = (wid * n_ch + c) * SC_CH
            pltpu.sync_copy(dst_hbm.at[pl.ds(i0, SC_CH)], idx_v)
            pltpu.async_copy(rows_hbm.at[idx_v], buf_v, sem).wait()
            pltpu.sync_copy(buf_v, oo_hbm.at[pl.ds(i0, SC_CH)])

    return k(rows, dst)


def _gmm(rows, be, w1, w2):
    NR, D = rows.shape
    E, F2, _ = w1.shape
    F = F2 // 2

    def kfn(be_ref, r_ref, w1_ref, w2_ref, o_ref):
        h = jnp.dot(r_ref[...], w1_ref[0].T, preferred_element_type=F32)
        a = h[:, :F]
        b2 = h[:, F:]
        g = a * jax.nn.sigmoid(a) * b2
        o_ref[...] = jnp.dot(g, w2_ref[0].T, preferred_element_type=F32)

    grid_spec = pltpu.PrefetchScalarGridSpec(
        num_scalar_prefetch=1,
        grid=(NB,),
        in_specs=[
            pl.BlockSpec((BM_G, D), lambda b, be: (b, 0)),
            pl.BlockSpec((1, F2, D), lambda b, be: (be[b], 0, 0)),
            pl.BlockSpec((1, D, F), lambda b, be: (be[b], 0, 0)),
        ],
        out_specs=pl.BlockSpec((BM_G, D), lambda b, be: (b, 0)),
    )
    return pl.pallas_call(
        kfn,
        grid_spec=grid_spec,
        out_shape=jax.ShapeDtypeStruct((NR, D), F32),
    )(be, rows, w1, w2)


def _combine(h, o1, o2, ws, bm=256):
    M, D = h.shape

    def kfn(h_ref, o1_ref, o2_ref, w_ref, out_ref):
        w = w_ref[...]
        out_ref[...] = (h_ref[...] + w[:, 0:1] * o1_ref[...]
                        + w[:, 1:2] * o2_ref[...])

    blk = pl.BlockSpec((bm, D), lambda i: (i, 0))
    return pl.pallas_call(
        kfn,
        grid=(M // bm,),
        in_specs=[blk, blk, blk, pl.BlockSpec((bm, 2), lambda i: (i, 0))],
        out_specs=blk,
        out_shape=jax.ShapeDtypeStruct((M, D), F32),
    )(h, o1, o2, ws)


def _moe_sparse(h, norm, gate, bias, w1, w2):
    M, D = h.shape
    xn, dst_c, ws, be = _route(h, norm, gate, bias)
    dst = dst_c.T.reshape(2 * M)
    rows = jnp.zeros((NROWS, D), F32).at[dst].set(jnp.concatenate([xn, xn], 0))
    rows_out = _gmm(rows, be.reshape(NB), w1, w2)
    oo = rows_out[dst]
    return _combine(h, oo[:M], oo[M:], ws)


def _heads(t):
    M = t.shape[0]
    return t.reshape(M, N_HEADS, HEAD_DIM).transpose(1, 0, 2)


def _unheads(t):
    H, M, Dh = t.shape
    return t.transpose(1, 0, 2).reshape(M, H * Dh)


def _enc_layer(h, p, ts):
    a = p['attn']
    w_cat = jnp.concatenate([a['w_qr'], a['w_kr'], a['w_kv_c']], axis=0)
    qkv = _linear(h, w_cat, norm=p['attn_norm'])
    q_r = qkv[:, :D_MODEL]
    k_r = qkv[:, D_MODEL:2 * D_MODEL]
    c_kv = qkv[:, 2 * D_MODEL:]
    up_cat = jnp.concatenate([a['w_kc_up'], a['w_vc_up']], axis=0)
    kv = _linear(c_kv, up_cat)
    k_c = kv[:, :D_MODEL]
    v_c = kv[:, D_MODEL:]
    o = _attn_rope(_heads(q_r), _heads(k_r), _heads(k_c), _heads(v_c), ts)
    h = _linear(_unheads(o), a['w_o'], res=h)
    return _moe_sparse(h, p['ffn_norm'], p['moe']['gate'], p['moe']['bias'],
                       p['moe']['w1'], p['moe']['w2'])


def _dec_layer(h, p, enc_kv):
    a = p['attn']
    w_cat = jnp.concatenate([a['w_qr'], a['w_kr'], a['w_kv_c']], axis=0)
    qkv = _linear(h, w_cat, norm=p['attn_norm'])
    q_r = qkv[:, :D_MODEL]
    k_r = qkv[:, D_MODEL:2 * D_MODEL]
    c_kv = qkv[:, 2 * D_MODEL:]
    up_cat = jnp.concatenate([a['w_kc_up'], a['w_vc_up']], axis=0)
    kv = _linear(c_kv, up_cat)
    k_sum = kv[:, :D_MODEL] + k_r
    v_c = kv[:, D_MODEL:]
    o = _attn_plain(_heads(q_r), _heads(k_sum), _heads(v_c), MAX_OUT)
    h = _linear(_unheads(o), a['w_o'], res=h)

    c = p['cross']
    wq = c['in_w'][:D_MODEL]
    bq = c['in_b'][:D_MODEL]
    qh = _linear(h, wq, b=bq, norm=p['cross_norm'])
    o = _attn_plain(_heads(qh), enc_kv[0], enc_kv[1], L)
    h = _linear(_unheads(o), c['out_w'], b=c['out_b'], res=h)

    return _moe(h, p['ffn_norm'], p['moe']['gate'], p['moe']['bias'],
                p['moe']['w1'], p['moe']['w2'])


def kernel(x, timestamps, params):
    B = x.shape[0]
    xf = x.reshape(-1, x.shape[-1])
    ts = timestamps.reshape(-1, 1)

    h = _linear(xf, params['in_W'], b=params['in_b'])
    for p in params['enc']:
        h = _enc_layer(h, p, ts)

    dec = jnp.zeros((DEC_PAD, D_MODEL), F32).at[:MAX_OUT].set(params['dec_query'])
    for p in params['dec']:
        c = p['cross']
        wkv = c['in_w'][D_MODEL:]
        bkv = c['in_b'][D_MODEL:]
        kvx = _linear(h, wkv, b=bkv)
        enc_kv = (_heads(kvx[:, :D_MODEL]), _heads(kvx[:, D_MODEL:]))
        dec = _dec_layer(dec, p, enc_kv)

    return dec[:MAX_OUT].reshape(B, MAX_OUT, D_MODEL)


# speedup vs baseline: 1.3257x; 1.3257x over previous
import functools
import math

import jax
import jax.numpy as jnp
from jax import lax
from jax.experimental import pallas as pl
from jax.experimental.pallas import tpu as pltpu, tpu_sc as plsc

L = 2048
D_MODEL = 768
N_HEADS = 12
HEAD_DIM = D_MODEL // N_HEADS
D_C = 128
D_FF = 1024
N_EXP = 8
MAX_OUT = 5
DEC_PAD = 8
THETA = 10000.0
EPS = 1e-6
F32 = jnp.float32


def _rms(x, w):
    return x * jax.lax.rsqrt(jnp.mean(x * x, axis=-1, keepdims=True) + EPS) * w


def _linear(x, w, b=None, res=None, norm=None, bm=None):
    M, K = x.shape
    N = w.shape[0]
    if bm is None:
        bm = 256 if M % 256 == 0 else M
    has_b, has_res, has_norm = b is not None, res is not None, norm is not None

    def kfn(*refs):
        idx = 2
        xv = refs[0][...]
        wv = refs[1][...]
        if has_norm:
            xv = _rms(xv, refs[idx][...])
            idx += 1
        y = jnp.dot(xv, wv.T, preferred_element_type=F32)
        if has_b:
            y = y + refs[idx][...]
            idx += 1
        if has_res:
            y = y + refs[idx][...]
            idx += 1
        refs[idx][...] = y

    inputs = [x, w]
    specs = [pl.BlockSpec((bm, K), lambda i: (i, 0)),
             pl.BlockSpec((N, K), lambda i: (0, 0))]
    if has_norm:
        inputs.append(norm.reshape(1, K))
        specs.append(pl.BlockSpec((1, K), lambda i: (0, 0)))
    if has_b:
        inputs.append(b.reshape(1, N))
        specs.append(pl.BlockSpec((1, N), lambda i: (0, 0)))
    if has_res:
        inputs.append(res)
        specs.append(pl.BlockSpec((bm, N), lambda i: (i, 0)))
    return pl.pallas_call(
        kfn,
        grid=(M // bm,),
        in_specs=specs,
        out_specs=pl.BlockSpec((bm, N), lambda i: (i, 0)),
        out_shape=jax.ShapeDtypeStruct((M, N), F32),
    )(*inputs)


def _attn_mla(qk, kv, ts=None, n_valid=None):
    Lq = qk.shape[0]
    Dh = HEAD_DIM
    scale = Dh ** -0.5
    use_rope = ts is not None

    def kfn(*refs):
        if use_rope:
            ts_ref, qk_ref, kv_ref, o_ref = refs
        else:
            qk_ref, kv_ref, o_ref = refs
        h = pl.program_id(0)
        half = Dh // 2

        if use_rope:
            idx = jax.lax.broadcasted_iota(jnp.int32, (1, half), 1).astype(F32)
            freqs = jnp.exp(-(math.log(THETA) * 2.0 / Dh) * idx)
            t = ts_ref[...] * freqs
            cs = jnp.cos(jnp.concatenate([t, t], axis=-1))

            def rope(v):
                rot = jnp.concatenate([-v[:, half:], v[:, :half]], axis=-1)
                return (v + rot) * cs
        else:
            rope = lambda v: v

        q = rope(qk_ref[:, :Dh])
        ks = kv_ref[:, :Dh] + rope(qk_ref[:, Dh:])
        s = jnp.dot(q * scale, ks.T, preferred_element_type=F32)
        if n_valid is not None and n_valid < Lq:
            kid = jax.lax.broadcasted_iota(jnp.int32, (Lq, Lq), 1)
            s = jnp.where(kid < n_valid, s, -1e30)
        m = jnp.max(s, axis=-1, keepdims=True)
        e = jnp.exp(s - m)
        r = 1.0 / jnp.sum(e, axis=-1, keepdims=True)
        out = jnp.dot(e, kv_ref[:, Dh:], preferred_element_type=F32) * r

        @pl.when(h % 2 == 0)
        def _():
            o_ref[:, :Dh] = out

        @pl.when(h % 2 == 1)
        def _():
            o_ref[:, Dh:] = out

    specs = [
        pl.BlockSpec((Lq, 2 * Dh), lambda h: (0, h)),
        pl.BlockSpec((Lq, 2 * Dh), lambda h: (0, h)),
    ]
    args = [qk, kv]
    if use_rope:
        specs.insert(0, pl.BlockSpec((Lq, 1), lambda h: (0, 0)))
        args.insert(0, ts)
    return pl.pallas_call(
        kfn,
        grid=(N_HEADS,),
        in_specs=specs,
        out_specs=pl.BlockSpec((Lq, 2 * Dh), lambda h: (0, h // 2)),
        out_shape=jax.ShapeDtypeStruct((Lq, D_MODEL), F32),
    )(*args)


def _attn_plain(q, kv, n_valid):
    Lq = q.shape[0]
    Lk = kv.shape[0]
    Dh = HEAD_DIM
    scale = Dh ** -0.5

    def kfn(q_ref, kv_ref, o_ref):
        h = pl.program_id(0)
        k = kv_ref[:, :Dh]
        v = kv_ref[:, Dh:]

        def one(qv):
            s = jnp.dot(qv * scale, k.T, preferred_element_type=F32)
            if n_valid < Lk:
                kid = jax.lax.broadcasted_iota(jnp.int32, (Lq, Lk), 1)
                s = jnp.where(kid < n_valid, s, -1e30)
            m = jnp.max(s, axis=-1, keepdims=True)
            e = jnp.exp(s - m)
            r = 1.0 / jnp.sum(e, axis=-1, keepdims=True)
            return jnp.dot(e, v, preferred_element_type=F32) * r

        @pl.when(h % 2 == 0)
        def _():
            o_ref[:, :Dh] = one(q_ref[:, :Dh])

        @pl.when(h % 2 == 1)
        def _():
            o_ref[:, Dh:] = one(q_ref[:, Dh:])

    return pl.pallas_call(
        kfn,
        grid=(N_HEADS,),
        in_specs=[
            pl.BlockSpec((Lq, 2 * Dh), lambda h: (0, h // 2)),
            pl.BlockSpec((Lk, 2 * Dh), lambda h: (0, h)),
        ],
        out_specs=pl.BlockSpec((Lq, 2 * Dh), lambda h: (0, h // 2)),
        out_shape=jax.ShapeDtypeStruct((Lq, D_MODEL), F32),
    )(q, kv)


def _moe(x, norm, gate, bias, w1, w2, bm=None):
    M, D = x.shape
    E, F2, _ = w1.shape
    F = F2 // 2
    if bm is None:
        bm = 256 if M % 256 == 0 else M

    def kfn(x_ref, n_ref, g_ref, b_ref, w1_ref, w2_ref, o_ref):
        e = pl.program_id(1)
        xv = x_ref[...]
        xn = _rms(xv, n_ref[...])
        logits = jnp.dot(xn, g_ref[...].T, preferred_element_type=F32) + b_ref[...]
        lm = jnp.max(logits, axis=-1, keepdims=True)
        ex = jnp.exp(logits - lm)
        probs = ex / jnp.sum(ex, axis=-1, keepdims=True)
        i1 = jnp.argmax(probs, axis=-1)
        m1 = jnp.max(probs, axis=-1)
        eid = jax.lax.broadcasted_iota(jnp.int32, probs.shape, 1)
        masked = jnp.where(eid == i1[:, None], -1.0, probs)
        i2 = jnp.argmax(masked, axis=-1)
        m2 = jnp.max(masked, axis=-1)
        wsum = m1 + m2
        we = (jnp.where(i1 == e, m1, 0.0) + jnp.where(i2 == e, m2, 0.0)) / wsum

        h = jnp.dot(xn, w1_ref[0].T, preferred_element_type=F32)
        a = h[:, :F]
        b2 = h[:, F:]
        g = a * jax.nn.sigmoid(a) * b2
        o = jnp.dot(g, w2_ref[0].T, preferred_element_type=F32)

        @pl.when(e == 0)
        def _():
            o_ref[...] = xv

        o_ref[...] += we[:, None] * o

    return pl.pallas_call(
        kfn,
        grid=(M // bm, E),
        in_specs=[
            pl.BlockSpec((bm, D), lambda i, e: (i, 0)),
            pl.BlockSpec((1, D), lambda i, e: (0, 0)),
            pl.BlockSpec((E, D), lambda i, e: (0, 0)),
            pl.BlockSpec((1, E), lambda i, e: (0, 0)),
            pl.BlockSpec((1, F2, D), lambda i, e: (e, 0, 0)),
            pl.BlockSpec((1, D, F), lambda i, e: (e, 0, 0)),
        ],
        out_specs=pl.BlockSpec((bm, D), lambda i, e: (i, 0)),
        out_shape=jax.ShapeDtypeStruct((M, D), F32),
    )(x, norm.reshape(1, D), gate, bias.reshape(1, E), w1, w2)


BM_G = 128
NROWS = 2 * L + N_EXP * BM_G
NB = NROWS // BM_G
SC_NW = 32
SC_CH = 64


def _route(h, norm, gate, bias):
    M, D = h.shape
    E = N_EXP
    BLK = 512
    NBLK = 2 * M // BLK

    def kfn(h_ref, n_ref, g_ref, b_ref, xn_ref, dst_ref, ws_ref, be_ref):
        xv = h_ref[...]
        xn = _rms(xv, n_ref[...])
        xn_ref[...] = xn
        logits = jnp.dot(xn, g_ref[...].T, preferred_element_type=F32, precision=lax.Precision.HIGHEST) + b_ref[...]
        lm = jnp.max(logits, axis=-1, keepdims=True)
        ex = jnp.exp(logits - lm)
        probs = ex / jnp.sum(ex, axis=-1, keepdims=True)
        i1 = jnp.argmax(probs, axis=-1)
        m1 = jnp.max(probs, axis=-1, keepdims=True)
        eid = jax.lax.broadcasted_iota(jnp.int32, probs.shape, 1)
        masked = jnp.where(eid == i1[:, None], -1.0, probs)
        i2 = jnp.argmax(masked, axis=-1)
        m2 = jnp.max(masked, axis=-1, keepdims=True)
        wsum = m1 + m2
        ws_ref[...] = jnp.concatenate([m1 / wsum, m2 / wsum], axis=1)

        one1 = (eid == i1[:, None]).astype(F32)
        one2 = (eid == i2[:, None]).astype(F32)
        ri = jax.lax.broadcasted_iota(jnp.int32, (BLK, BLK), 0)
        ci = jax.lax.broadcasted_iota(jnp.int32, (BLK, BLK), 1)
        tri = (ci < ri).astype(F32)
        ones_e = jnp.ones((E, 1), F32)

        carry = jnp.zeros((1, E), F32)
        ranks, onehots = [], []
        for one in (one1, one2):
            for b in range(M // BLK):
                ob = one[b * BLK:(b + 1) * BLK]
                r = jnp.dot(tri, ob, preferred_element_type=F32) + carry
                ranks.append(jnp.sum(r * ob, axis=1, keepdims=True))
                onehots.append(ob)
                carry = carry + jnp.sum(ob, axis=0, keepdims=True)

        counts = carry.astype(jnp.int32)
        padded = ((counts + BM_G - 1) // BM_G) * BM_G
        padded_f = padded.astype(F32)
        ji = jax.lax.broadcasted_iota(jnp.int32, (E, E), 0)
        ei = jax.lax.broadcasted_iota(jnp.int32, (E, E), 1)
        u_strict = (ji < ei).astype(F32)
        u_incl = (ji <= ei).astype(F32)
        rstart = jnp.dot(padded_f, u_strict, preferred_element_type=F32, precision=lax.Precision.HIGHEST)
        cumb = jnp.dot(padded_f / BM_G, u_incl, preferred_element_type=F32, precision=lax.Precision.HIGHEST)

        cols = []
        for ob, rk in zip(onehots, ranks):
            rsel = jnp.sum(ob * rstart, axis=1, keepdims=True)
            cols.append((rsel + rk).astype(jnp.int32))
        dst_ref[...] = jnp.concatenate(cols, axis=1)

        bix = jax.lax.broadcasted_iota(jnp.int32, (NB, 1), 0).astype(F32)
        ge = (bix >= cumb).astype(F32)
        nbefore = jnp.sum(ge, axis=1, keepdims=True)
        be_ref[...] = jnp.minimum(nbefore, E - 1).astype(jnp.int32)

    full = lambda shape: pl.BlockSpec(shape, lambda: tuple(0 for _ in shape))
    return pl.pallas_call(
        kfn,
        in_specs=[full((M, D)), full((1, D)), full((E, D)), full((1, E))],
        out_specs=[full((M, D)), full((BLK, NBLK)), full((M, 2)), full((NB, 1))],
        out_shape=[
            jax.ShapeDtypeStruct((M, D), F32),
            jax.ShapeDtypeStruct((BLK, NBLK), jnp.int32),
            jax.ShapeDtypeStruct((M, 2), F32),
            jax.ShapeDtypeStruct((NB, 1), jnp.int32),
        ],
    )(h, norm.reshape(1, D), gate, bias.reshape(1, E))


def _sc_scatter_rows(xn, dst):
    M, D = xn.shape
    mesh = plsc.VectorSubcoreMesh(core_axis_name="c", subcore_axis_name="s")
    n_ch = 2 * M // (SC_NW * SC_CH)

    @functools.partial(
        pl.kernel, mesh=mesh,
        out_type=jax.ShapeDtypeStruct((NROWS, D), F32),
        scratch_types=[
            pltpu.VMEM((SC_CH,), jnp.int32),
            pltpu.VMEM((SC_CH, D), F32),
            pltpu.SemaphoreType.DMA,
        ],
    )
    def k(xn_hbm, dst_hbm, rows_hbm, idx_v, buf_v, sem):
        wid = lax.axis_index("s") * 2 + lax.axis_index("c")
        for c in range(n_ch):
            i0 = (wid * n_ch + c) * SC_CH
            tok0 = lax.rem(i0, M)
            pltpu.sync_copy(dst_hbm.at[pl.ds(i0, SC_CH)], idx_v)
            pltpu.sync_copy(xn_hbm.at[pl.ds(tok0, SC_CH)], buf_v)
            pltpu.async_copy(buf_v, rows_hbm.at[idx_v], sem).wait()

    return k(xn, dst)


def _sc_gather_rows(rows, dst):
    NR, D = rows.shape
    M2 = dst.shape[0]
    mesh = plsc.VectorSubcoreMesh(core_axis_name="c", subcore_axis_name="s")
    n_ch = M2 // (SC_NW * SC_CH)

    @functools.partial(
        pl.kernel, mesh=mesh,
        out_type=jax.ShapeDtypeStruct((M2, D), F32),
        scratch_types=[
            pltpu.VMEM((SC_CH,), jnp.int32),
            pltpu.VMEM((SC_CH, D), F32),
            pltpu.SemaphoreType.DMA,
        ],
    )
    def k(rows_hbm, dst_hbm, oo_hbm, idx_v, buf_v, sem):
        wid = lax.axis_index("s") * 2 + lax.axis_index("c")
        for c in range(n_ch):
            i0 = (wid * n_ch + c) * SC_CH
            pltpu.sync_copy(dst_hbm.at[pl.ds(i0, SC_CH)], idx_v)
            pltpu.async_copy(rows_hbm.at[idx_v], buf_v, sem).wait()
            pltpu.sync_copy(buf_v, oo_hbm.at[pl.ds(i0, SC_CH)])

    return k(rows, dst)


def _gmm(rows, be, w1, w2):
    NR, D = rows.shape
    E, F2, _ = w1.shape
    F = F2 // 2

    def kfn(be_ref, r_ref, w1_ref, w2_ref, o_ref):
        h = jnp.dot(r_ref[...], w1_ref[0].T, preferred_element_type=F32)
        a = h[:, :F]
        b2 = h[:, F:]
        g = a * jax.nn.sigmoid(a) * b2
        o_ref[...] = jnp.dot(g, w2_ref[0].T, preferred_element_type=F32)

    grid_spec = pltpu.PrefetchScalarGridSpec(
        num_scalar_prefetch=1,
        grid=(NB,),
        in_specs=[
            pl.BlockSpec((BM_G, D), lambda b, be: (b, 0)),
            pl.BlockSpec((1, F2, D), lambda b, be: (be[b], 0, 0)),
            pl.BlockSpec((1, D, F), lambda b, be: (be[b], 0, 0)),
        ],
        out_specs=pl.BlockSpec((BM_G, D), lambda b, be: (b, 0)),
    )
    return pl.pallas_call(
        kfn,
        grid_spec=grid_spec,
        out_shape=jax.ShapeDtypeStruct((NR, D), F32),
    )(be, rows, w1, w2)


def _combine(h, o1, o2, ws, bm=256):
    M, D = h.shape

    def kfn(h_ref, o1_ref, o2_ref, w_ref, out_ref):
        w = w_ref[...]
        out_ref[...] = (h_ref[...] + w[:, 0:1] * o1_ref[...]
                        + w[:, 1:2] * o2_ref[...])

    blk = pl.BlockSpec((bm, D), lambda i: (i, 0))
    return pl.pallas_call(
        kfn,
        grid=(M // bm,),
        in_specs=[blk, blk, blk, pl.BlockSpec((bm, 2), lambda i: (i, 0))],
        out_specs=blk,
        out_shape=jax.ShapeDtypeStruct((M, D), F32),
    )(h, o1, o2, ws)


def _moe_sparse(h, norm, gate, bias, w1, w2):
    M, D = h.shape
    xn, dst_c, ws, be = _route(h, norm, gate, bias)
    dst = dst_c.T.reshape(2 * M)
    rows = _sc_scatter_rows(xn, dst)
    rows_out = _gmm(rows, be.reshape(NB), w1, w2)
    oo = _sc_gather_rows(rows_out, dst)
    return _combine(h, oo[:M], oo[M:], ws)


def _ilv(wa, wb):
    K = wa.shape[1]
    a = wa.reshape(N_HEADS, HEAD_DIM, K)
    b = wb.reshape(N_HEADS, HEAD_DIM, K)
    return jnp.concatenate([a, b], axis=1).reshape(2 * N_HEADS * HEAD_DIM, K)


def _ilv_b(ba, bb):
    a = ba.reshape(N_HEADS, HEAD_DIM)
    b = bb.reshape(N_HEADS, HEAD_DIM)
    return jnp.concatenate([a, b], axis=1).reshape(2 * N_HEADS * HEAD_DIM)


def _mla_qkv(h, p, norm):
    a = p['attn']
    w_cat = jnp.concatenate([_ilv(a['w_qr'], a['w_kr']), a['w_kv_c']], axis=0)
    qkv = _linear(h, w_cat, norm=norm)
    kv = _linear(qkv[:, 2 * D_MODEL:], _ilv(a['w_kc_up'], a['w_vc_up']))
    return qkv, kv


def _enc_layer(h, p, ts):
    qkv, kv = _mla_qkv(h, p, p['attn_norm'])
    o = _attn_mla(qkv, kv, ts=ts)
    h = _linear(o, p['attn']['w_o'], res=h)
    return _moe_sparse(h, p['ffn_norm'], p['moe']['gate'], p['moe']['bias'],
                       p['moe']['w1'], p['moe']['w2'])


def _dec_layer(h, p, enc_kv):
    qkv, kv = _mla_qkv(h, p, p['attn_norm'])
    o = _attn_mla(qkv, kv, n_valid=MAX_OUT)
    h = _linear(o, p['attn']['w_o'], res=h)

    c = p['cross']
    qh = _linear(h, c['in_w'][:D_MODEL], b=c['in_b'][:D_MODEL],
                 norm=p['cross_norm'])
    o = _attn_plain(qh, enc_kv, L)
    h = _linear(o, c['out_w'], b=c['out_b'], res=h)

    return _moe(h, p['ffn_norm'], p['moe']['gate'], p['moe']['bias'],
                p['moe']['w1'], p['moe']['w2'])


def kernel(x, timestamps, params):
    B = x.shape[0]
    xf = x.reshape(-1, x.shape[-1])
    ts = timestamps.reshape(-1, 1)

    h = _linear(xf, params['in_W'], b=params['in_b'])
    for p in params['enc']:
        h = _enc_layer(h, p, ts)

    dec = jnp.zeros((DEC_PAD, D_MODEL), F32).at[:MAX_OUT].set(params['dec_query'])
    for p in params['dec']:
        c = p['cross']
        wkv = _ilv(c['in_w'][D_MODEL:2 * D_MODEL], c['in_w'][2 * D_MODEL:])
        bkv = _ilv_b(c['in_b'][D_MODEL:2 * D_MODEL], c['in_b'][2 * D_MODEL:])
        enc_kv = _linear(h, wkv, b=bkv)
        dec = _dec_layer(dec, p, enc_kv)

    return dec[:MAX_OUT].reshape(B, MAX_OUT, D_MODEL)
